# baseline (speedup 1.0000x reference)
import functools

import jax
import jax.numpy as jnp
from jax import lax
from jax.experimental import pallas as pl
from jax.experimental.pallas import tpu as pltpu
from jax.experimental.pallas import tpu_sc as plsc

T = 16384
K = 7
NE = 64
INV_ROUTED_SCALING = 1.0 / 1.5
TPW = 512

_P = (0.9999819, -0.49918786, 0.3244118, -0.20866966, 0.10028721,
      -0.023689253)

try:
    _INFO = plsc.get_sparse_core_info()
    _NC = _INFO.num_cores
    _NS = _INFO.num_subcores
except Exception:
    _NC, _NS = 2, 16


def _sqrt_softplus(x):
    e = jnp.exp(-jnp.abs(x))
    acc = jnp.full((16,), _P[5], jnp.float32)
    for k in range(4, -1, -1):
        acc = acc * e + _P[k]
    sp = jnp.maximum(x, 0.0) + acc * e
    b = plsc.bitcast(sp, jnp.int32)
    q = plsc.bitcast(0x5F3759DF - lax.shift_right_logical(b, 1), jnp.float32)
    for _ in range(3):
        q = q * (1.5 - 0.5 * sp * q * q)
    return sp * q


@functools.cache
def _build_route():
    @functools.partial(
        pl.kernel,
        mesh=plsc.VectorSubcoreMesh(core_axis_name="c", subcore_axis_name="s"),
        out_type=[
            jax.ShapeDtypeStruct((T, K + 1), jnp.float32),
            jax.ShapeDtypeStruct((T, K + 1), jnp.int32),
        ],
        scratch_types=[
            pltpu.VMEM((TPW,), jnp.int32),
            pltpu.VMEM((K, 4, 128), jnp.int32),
            pltpu.VMEM((K, 4, 128), jnp.int32),
            pltpu.VMEM((TPW * NE,), jnp.float32),
            pltpu.VMEM((TPW, K + 1), jnp.float32),
            pltpu.VMEM((TPW, K + 1), jnp.int32),
            pltpu.SemaphoreType.DMA,
            pltpu.SemaphoreType.DMA,
        ],
        compiler_params=pltpu.CompilerParams(
            use_tc_tiling_on_sc=False, needs_layout_passes=False),
    )
    def _route(ids_hbm, tab_hbm, lg_hbm, w_hbm, i_hbm,
               idx_v, widx_v, eid_v, lg_v, ow_v, oi_v, sem, sem2):
        wid = lax.axis_index("s") * _NC + lax.axis_index("c")
        base = wid * TPW
        lg_cp = pltpu.async_copy(
            lg_hbm.at[pl.ds(base * NE, TPW * NE)], lg_v, sem2)
        pltpu.sync_copy(ids_hbm.at[pl.ds(base, TPW)], idx_v)

        for c in range(4):
            def wbody(g8, carry, c=c):
                tok7 = idx_v[pl.ds(c * 128 + g8 * 16, 16)] * 7
                for j in range(K):
                    widx_v[j, c, pl.ds(g8 * 16, 16)] = tok7 + j
                return carry
            lax.fori_loop(0, 8, wbody, 0)

        cps = [pltpu.async_copy(tab_hbm.at[widx_v.at[j, c]],
                                eid_v.at[j, c], sem)
               for j in range(K) for c in range(4)]
        for cp in cps:
            cp.wait()
        lg_cp.wait()

        for c in range(4):
            def ebody(g8, carry, c=c):
                t16 = jnp.arange(16, dtype=jnp.int32) + c * 128 + g8 * 16
                lbase = t16 * NE
                eids, ws = [], []
                for j in range(K):
                    eid = eid_v[j, c, pl.ds(g8 * 16, 16)]
                    x = plsc.load_gather(lg_v, [lbase + eid])
                    eids.append(eid)
                    ws.append(_sqrt_softplus(x))
                wsum = ws[0]
                for w in ws[1:]:
                    wsum = wsum + w
                inv = 1.0 / wsum
                sn = jnp.zeros((16,), jnp.float32)
                for j in range(K):
                    wn = ws[j] * inv
                    sn = sn + wn
                    cj = jnp.full((16,), j, jnp.int32)
                    plsc.store_scatter(ow_v, [t16, cj], wn)
                    plsc.store_scatter(oi_v, [t16, cj], eids[j])
                c7 = jnp.full((16,), K, jnp.int32)
                plsc.store_scatter(ow_v, [t16, c7], sn * INV_ROUTED_SCALING)
                plsc.store_scatter(oi_v, [t16, c7],
                                   jnp.full((16,), NE, jnp.int32))
                return carry
            lax.fori_loop(0, 8, ebody, 0)

        pltpu.sync_copy(ow_v, w_hbm.at[pl.ds(base, TPW)])
        pltpu.sync_copy(oi_v, i_hbm.at[pl.ds(base, TPW)])

    return _route


def kernel(hidden_states, router_logits, input_ids, tid2eid):
    del hidden_states
    tab_flat = tid2eid.reshape(100000 * K)
    lg_flat = router_logits.reshape(T * NE)
    topk_weights, topk_ids = _build_route()(input_ids, tab_flat, lg_flat)
    return topk_weights, topk_ids, router_logits


# speedup vs baseline: 1.1266x; 1.1266x over previous
import functools

import jax
import jax.numpy as jnp
from jax import lax
from jax.experimental import pallas as pl
from jax.experimental.pallas import tpu as pltpu
from jax.experimental.pallas import tpu_sc as plsc

T = 16384
K = 7
NE = 64
INV_ROUTED_SCALING = 1.0 / 1.5
TPW = 512
CH = 128

_P = (0.9999819, -0.49918786, 0.3244118, -0.20866966, 0.10028721,
      -0.023689253)

try:
    _INFO = plsc.get_sparse_core_info()
    _NC = _INFO.num_cores
    _NS = _INFO.num_subcores
except Exception:
    _NC, _NS = 2, 16


def _sqrt_softplus(x):
    e = jnp.exp(-jnp.abs(x))
    acc = jnp.full((16,), _P[5], jnp.float32)
    for k in range(4, -1, -1):
        acc = acc * e + _P[k]
    sp = jnp.maximum(x, 0.0) + acc * e
    b = plsc.bitcast(sp, jnp.int32)
    q = plsc.bitcast(0x5F3759DF - lax.shift_right_logical(b, 1), jnp.float32)
    for _ in range(3):
        q = q * (1.5 - 0.5 * sp * q * q)
    return sp * q


@functools.cache
def _build_route():
    @functools.partial(
        pl.kernel,
        mesh=plsc.VectorSubcoreMesh(core_axis_name="c", subcore_axis_name="s"),
        out_type=[
            jax.ShapeDtypeStruct((T, K + 1), jnp.float32),
            jax.ShapeDtypeStruct((T, K + 1), jnp.int32),
        ],
        scratch_types=[
            pltpu.VMEM((TPW,), jnp.int32),
            pltpu.VMEM((CH, 128), jnp.int32),
            pltpu.VMEM((CH, NE), jnp.float32),
            pltpu.VMEM((CH, K + 1), jnp.float32),
            pltpu.VMEM((CH, K + 1), jnp.int32),
            pltpu.SemaphoreType.DMA,
            pltpu.SemaphoreType.DMA,
        ],
        compiler_params=pltpu.CompilerParams(needs_layout_passes=False),
    )
    def _route(ids_hbm, tab_hbm, lg_hbm, w_hbm, i_hbm,
               idx_v, win_v, lg_v, ow_v, oi_v, sem, sem2):
        wid = lax.axis_index("s") * _NC + lax.axis_index("c")
        base = wid * TPW
        pltpu.sync_copy(ids_hbm.at[pl.ds(base, TPW)], idx_v)

        for c in range(4):
            r0 = base + c * CH
            g_cp = pltpu.async_copy(
                tab_hbm.at[idx_v.at[pl.ds(c * CH, CH)]], win_v, sem)
            l_cp = pltpu.async_copy(lg_hbm.at[pl.ds(r0, CH)], lg_v, sem2)
            g_cp.wait()
            l_cp.wait()

            def ebody(g8, carry):
                t16 = jnp.arange(16, dtype=jnp.int32) + g8 * 16
                eids, ws = [], []
                for j in range(K):
                    eid = plsc.load_gather(
                        win_v, [t16, jnp.full((16,), j, jnp.int32)])
                    x = plsc.load_gather(lg_v, [t16, eid])
                    eids.append(eid)
                    ws.append(_sqrt_softplus(x))
                wsum = ws[0]
                for w in ws[1:]:
                    wsum = wsum + w
                inv = 1.0 / wsum
                sn = jnp.zeros((16,), jnp.float32)
                for j in range(K):
                    wn = ws[j] * inv
                    sn = sn + wn
                    cj = jnp.full((16,), j, jnp.int32)
                    plsc.store_scatter(ow_v, [t16, cj], wn)
                    plsc.store_scatter(oi_v, [t16, cj], eids[j])
                c7 = jnp.full((16,), K, jnp.int32)
                plsc.store_scatter(ow_v, [t16, c7], sn * INV_ROUTED_SCALING)
                plsc.store_scatter(oi_v, [t16, c7],
                                   jnp.full((16,), NE, jnp.int32))
                return carry

            lax.fori_loop(0, CH // 16, ebody, 0)

            pltpu.sync_copy(ow_v, w_hbm.at[pl.ds(r0, CH)])
            pltpu.sync_copy(oi_v, i_hbm.at[pl.ds(r0, CH)])

    return _route


def kernel(hidden_states, router_logits, input_ids, tid2eid):
    del hidden_states
    tab128 = jnp.pad(tid2eid, ((0, 0), (0, 128 - K)))
    topk_weights, topk_ids = _build_route()(input_ids, tab128, router_logits)
    return topk_weights, topk_ids, router_logits


# speedup vs baseline: 1.8105x; 1.6071x over previous
import functools

import jax
import jax.numpy as jnp
from jax import lax
from jax.experimental import pallas as pl
from jax.experimental.pallas import tpu as pltpu
from jax.experimental.pallas import tpu_sc as plsc

T = 16384
K = 7
NE = 64
V = 100000
INV_ROUTED_SCALING = 1.0 / 1.5
TPW = 512

_P = (0.9999819, -0.49918786, 0.3244118, -0.20866966, 0.10028721,
      -0.023689253)

try:
    _INFO = plsc.get_sparse_core_info()
    _NC = _INFO.num_cores
    _NS = _INFO.num_subcores
except Exception:
    _NC, _NS = 2, 16


def _sqrt_softplus(x):
    e = jnp.exp(-jnp.abs(x))
    acc = jnp.full((16,), _P[5], jnp.float32)
    for k in range(4, -1, -1):
        acc = acc * e + _P[k]
    sp = jnp.maximum(x, 0.0) + acc * e
    b = plsc.bitcast(sp, jnp.int32)
    q = plsc.bitcast(0x5F3759DF - lax.shift_right_logical(b, 1), jnp.float32)
    for _ in range(3):
        q = q * (1.5 - 0.5 * sp * q * q)
    return sp * q


@functools.cache
def _build_route():
    @functools.partial(
        pl.kernel,
        mesh=plsc.VectorSubcoreMesh(core_axis_name="c", subcore_axis_name="s"),
        out_type=[
            jax.ShapeDtypeStruct((T, K + 1), jnp.float32),
            jax.ShapeDtypeStruct((T, K + 1), jnp.int32),
        ],
        scratch_types=[
            pltpu.VMEM((TPW,), jnp.int32),
            pltpu.VMEM((K, 4, 128), jnp.int32),
            pltpu.VMEM((K, 4, 128), jnp.int32),
            pltpu.VMEM((TPW * NE,), jnp.float32),
            pltpu.VMEM((TPW, K + 1), jnp.float32),
            pltpu.VMEM((TPW, K + 1), jnp.int32),
            pltpu.SemaphoreType.DMA,
            pltpu.SemaphoreType.DMA,
        ],
        compiler_params=pltpu.CompilerParams(
            use_tc_tiling_on_sc=False, needs_layout_passes=False),
    )
    def _route(ids_hbm, tab_hbm, lg_hbm, w_hbm, i_hbm,
               idx_v, widx_v, eid_v, lg_v, ow_v, oi_v, sem, sem2):
        wid = lax.axis_index("s") * _NC + lax.axis_index("c")
        base = wid * TPW
        lg_cp = pltpu.async_copy(
            lg_hbm.at[pl.ds(base * NE, TPW * NE)], lg_v, sem2)
        pltpu.sync_copy(ids_hbm.at[pl.ds(base, TPW)], idx_v)

        for c in range(4):
            def wbody(g8, carry, c=c):
                tok = idx_v[pl.ds(c * 128 + g8 * 16, 16)]
                for j in range(K):
                    widx_v[j, c, pl.ds(g8 * 16, 16)] = tok + j * V
                return carry
            lax.fori_loop(0, 8, wbody, 0)

        cps = [pltpu.async_copy(tab_hbm.at[widx_v.at[j, c]],
                                eid_v.at[j, c], sem)
               for j in range(K) for c in range(4)]
        for cp in cps:
            cp.wait()
        lg_cp.wait()

        for c in range(4):
            def ebody(g8, carry, c=c):
                t16 = jnp.arange(16, dtype=jnp.int32) + c * 128 + g8 * 16
                lbase = t16 * NE
                eids, ws = [], []
                for j in range(K):
                    eid = eid_v[j, c, pl.ds(g8 * 16, 16)]
                    x = plsc.load_gather(lg_v, [lbase + eid])
                    eids.append(eid)
                    ws.append(_sqrt_softplus(x))
                wsum = ws[0]
                for w in ws[1:]:
                    wsum = wsum + w
                inv = 1.0 / wsum
                sn = jnp.zeros((16,), jnp.float32)
                for j in range(K):
                    wn = ws[j] * inv
                    sn = sn + wn
                    cj = jnp.full((16,), j, jnp.int32)
                    plsc.store_scatter(ow_v, [t16, cj], wn)
                    plsc.store_scatter(oi_v, [t16, cj], eids[j])
                c7 = jnp.full((16,), K, jnp.int32)
                plsc.store_scatter(ow_v, [t16, c7], sn * INV_ROUTED_SCALING)
                plsc.store_scatter(oi_v, [t16, c7],
                                   jnp.full((16,), NE, jnp.int32))
                return carry
            lax.fori_loop(0, 8, ebody, 0)

        pltpu.sync_copy(ow_v, w_hbm.at[pl.ds(base, TPW)])
        pltpu.sync_copy(oi_v, i_hbm.at[pl.ds(base, TPW)])

    return _route


def kernel(hidden_states, router_logits, input_ids, tid2eid):
    del hidden_states
    tab_flat = tid2eid.T.reshape(V * K)
    lg_flat = router_logits.reshape(T * NE)
    topk_weights, topk_ids = _build_route()(input_ids, tab_flat, lg_flat)
    return topk_weights, topk_ids, router_logits


# speedup vs baseline: 2.7686x; 1.5292x over previous
import functools

import jax
import jax.numpy as jnp
from jax import lax
from jax.experimental import pallas as pl
from jax.experimental.pallas import tpu as pltpu
from jax.experimental.pallas import tpu_sc as plsc

T = 16384
K = 7
NE = 64
V = 100000
INV_ROUTED_SCALING = 1.0 / 1.5
TPW = 512

_P = (0.9999819, -0.49918786, 0.3244118, -0.20866966, 0.10028721,
      -0.023689253)

try:
    _INFO = plsc.get_sparse_core_info()
    _NC = _INFO.num_cores
    _NS = _INFO.num_subcores
except Exception:
    _NC, _NS = 2, 16


def _sqrt_softplus(x):
    e = jnp.exp(-jnp.abs(x))
    acc = jnp.full((16,), _P[5], jnp.float32)
    for k in range(4, -1, -1):
        acc = acc * e + _P[k]
    sp = jnp.maximum(x, 0.0) + acc * e
    b = plsc.bitcast(sp, jnp.int32)
    q = plsc.bitcast(0x5F3759DF - lax.shift_right_logical(b, 1), jnp.float32)
    for _ in range(3):
        q = q * (1.5 - 0.5 * sp * q * q)
    return sp * q


@functools.cache
def _build_route():
    @functools.partial(
        pl.kernel,
        mesh=plsc.VectorSubcoreMesh(core_axis_name="c", subcore_axis_name="s"),
        out_type=[
            jax.ShapeDtypeStruct((T // 128, K + 1, 128), jnp.float32),
            jax.ShapeDtypeStruct((T // 128, K + 1, 128), jnp.int32),
        ],
        scratch_types=[
            pltpu.VMEM((TPW,), jnp.int32),
            pltpu.VMEM((K, 4, 128), jnp.int32),
            pltpu.VMEM((K, 4, 128), jnp.int32),
            pltpu.VMEM((TPW * NE,), jnp.float32),
            pltpu.VMEM((4, K + 1, 128), jnp.float32),
            pltpu.VMEM((4, K + 1, 128), jnp.int32),
            pltpu.SemaphoreType.DMA,
            pltpu.SemaphoreType.DMA,
        ],
        compiler_params=pltpu.CompilerParams(
            use_tc_tiling_on_sc=False, needs_layout_passes=False),
    )
    def _route(ids_hbm, tab_hbm, lg_hbm, w_hbm, i_hbm,
               idx_v, widx_v, eid_v, lg_v, ow_v, oi_v, sem, sem2):
        wid = lax.axis_index("s") * _NC + lax.axis_index("c")
        base = wid * TPW
        lg_cp = pltpu.async_copy(
            lg_hbm.at[pl.ds(base * NE, TPW * NE)], lg_v, sem2)
        pltpu.sync_copy(ids_hbm.at[pl.ds(base, TPW)], idx_v)

        for c in range(4):
            def wbody(g8, carry, c=c):
                tok = idx_v[pl.ds(c * 128 + g8 * 16, 16)]
                for j in range(K):
                    widx_v[j, c, pl.ds(g8 * 16, 16)] = tok + j * V
                return carry
            lax.fori_loop(0, 8, wbody, 0)

        cps = [pltpu.async_copy(tab_hbm.at[widx_v.at[j, c]],
                                eid_v.at[j, c], sem)
               for j in range(K) for c in range(4)]
        for cp in cps:
            cp.wait()
        lg_cp.wait()

        for c in range(4):
            def ebody(g8, carry, c=c):
                t16 = jnp.arange(16, dtype=jnp.int32) + c * 128 + g8 * 16
                lbase = t16 * NE
                eids, ws = [], []
                for j in range(K):
                    eid = eid_v[j, c, pl.ds(g8 * 16, 16)]
                    x = plsc.load_gather(lg_v, [lbase + eid])
                    eids.append(eid)
                    ws.append(_sqrt_softplus(x))
                wsum = ws[0]
                for w in ws[1:]:
                    wsum = wsum + w
                inv = 1.0 / wsum
                sn = jnp.zeros((16,), jnp.float32)
                for j in range(K):
                    wn = ws[j] * inv
                    sn = sn + wn
                    ow_v[c, j, pl.ds(g8 * 16, 16)] = wn
                    oi_v[c, j, pl.ds(g8 * 16, 16)] = eids[j]
                ow_v[c, K, pl.ds(g8 * 16, 16)] = sn * INV_ROUTED_SCALING
                oi_v[c, K, pl.ds(g8 * 16, 16)] = jnp.full((16,), NE,
                                                          jnp.int32)
                return carry
            lax.fori_loop(0, 8, ebody, 0)

        pltpu.sync_copy(ow_v, w_hbm.at[pl.ds(base // 128, 4)])
        pltpu.sync_copy(oi_v, i_hbm.at[pl.ds(base // 128, 4)])

    return _route


def kernel(hidden_states, router_logits, input_ids, tid2eid):
    del hidden_states
    tab_flat = tid2eid.T.reshape(V * K)
    lg_flat = router_logits.reshape(T * NE)
    wbuf, ibuf = _build_route()(input_ids, tab_flat, lg_flat)
    topk_weights = wbuf.transpose(0, 2, 1).reshape(T, K + 1)
    topk_ids = ibuf.transpose(0, 2, 1).reshape(T, K + 1)
    return topk_weights, topk_ids, router_logits


# speedup vs baseline: 2.8051x; 1.0132x over previous
import functools

import jax
import jax.numpy as jnp
from jax import lax
from jax.experimental import pallas as pl
from jax.experimental.pallas import tpu as pltpu
from jax.experimental.pallas import tpu_sc as plsc

T = 16384
K = 7
NE = 64
V = 100000
INV_ROUTED_SCALING = 1.0 / 1.5
TPW = 512

_P = (0.9998879, -0.49636775, 0.30467087, -0.15602694, 0.041064072)

try:
    _INFO = plsc.get_sparse_core_info()
    _NC = _INFO.num_cores
    _NS = _INFO.num_subcores
except Exception:
    _NC, _NS = 2, 16


def _sqrt_softplus(x):
    e = jnp.exp(-jnp.abs(x))
    acc = jnp.full((16,), _P[4], jnp.float32)
    for k in range(3, -1, -1):
        acc = acc * e + _P[k]
    sp = jnp.maximum(x, 0.0) + acc * e
    b = plsc.bitcast(sp, jnp.int32)
    q = plsc.bitcast(0x5F3759DF - lax.shift_right_logical(b, 1), jnp.float32)
    for _ in range(2):
        q = q * (1.5 - 0.5 * sp * q * q)
    return sp * q


@functools.cache
def _build_route():
    @functools.partial(
        pl.kernel,
        mesh=plsc.VectorSubcoreMesh(core_axis_name="c", subcore_axis_name="s"),
        out_type=[
            jax.ShapeDtypeStruct((T // 128, K + 1, 128), jnp.float32),
            jax.ShapeDtypeStruct((T // 128, K + 1, 128), jnp.int32),
        ],
        scratch_types=[
            pltpu.VMEM((TPW,), jnp.int32),
            pltpu.VMEM((K, 4, 128), jnp.int32),
            pltpu.VMEM((K, 4, 128), jnp.int32),
            pltpu.VMEM((TPW * NE,), jnp.float32),
            pltpu.VMEM((4, K + 1, 128), jnp.float32),
            pltpu.VMEM((4, K + 1, 128), jnp.int32),
            pltpu.SemaphoreType.DMA,
            pltpu.SemaphoreType.DMA,
        ],
        compiler_params=pltpu.CompilerParams(
            use_tc_tiling_on_sc=False, needs_layout_passes=False),
    )
    def _route(ids_hbm, tab_hbm, lg_hbm, w_hbm, i_hbm,
               idx_v, widx_v, eid_v, lg_v, ow_v, oi_v, sem, sem2):
        wid = lax.axis_index("s") * _NC + lax.axis_index("c")
        base = wid * TPW
        lg_cp = pltpu.async_copy(
            lg_hbm.at[pl.ds(base * NE, TPW * NE)], lg_v, sem2)
        pltpu.sync_copy(ids_hbm.at[pl.ds(base, TPW)], idx_v)

        for c in range(4):
            def wbody(g8, carry, c=c):
                tok = idx_v[pl.ds(c * 128 + g8 * 16, 16)]
                for j in range(K):
                    widx_v[j, c, pl.ds(g8 * 16, 16)] = tok + j * V
                return carry
            lax.fori_loop(0, 8, wbody, 0)

        cps = [[pltpu.async_copy(tab_hbm.at[widx_v.at[j, c]],
                                 eid_v.at[j, c], sem)
                for j in range(K)] for c in range(4)]
        lg_cp.wait()

        for c in range(4):
            for cp in cps[c]:
                cp.wait()

            def ebody(g8, carry, c=c):
                t16 = jnp.arange(16, dtype=jnp.int32) + c * 128 + g8 * 16
                lbase = t16 * NE
                eids, ws = [], []
                for j in range(K):
                    eid = eid_v[j, c, pl.ds(g8 * 16, 16)]
                    x = plsc.load_gather(lg_v, [lbase + eid])
                    eids.append(eid)
                    ws.append(_sqrt_softplus(x))
                wsum = ws[0]
                for w in ws[1:]:
                    wsum = wsum + w
                inv = 1.0 / wsum
                sn = jnp.zeros((16,), jnp.float32)
                for j in range(K):
                    wn = ws[j] * inv
                    sn = sn + wn
                    ow_v[c, j, pl.ds(g8 * 16, 16)] = wn
                    oi_v[c, j, pl.ds(g8 * 16, 16)] = eids[j]
                ow_v[c, K, pl.ds(g8 * 16, 16)] = sn * INV_ROUTED_SCALING
                oi_v[c, K, pl.ds(g8 * 16, 16)] = jnp.full((16,), NE,
                                                          jnp.int32)
                return carry
            lax.fori_loop(0, 8, ebody, 0)

        pltpu.sync_copy(ow_v, w_hbm.at[pl.ds(base // 128, 4)])
        pltpu.sync_copy(oi_v, i_hbm.at[pl.ds(base // 128, 4)])

    return _route


def kernel(hidden_states, router_logits, input_ids, tid2eid):
    del hidden_states
    tab_flat = tid2eid.T.reshape(V * K)
    lg_flat = router_logits.reshape(T * NE)
    wbuf, ibuf = _build_route()(input_ids, tab_flat, lg_flat)
    topk_weights = wbuf.transpose(0, 2, 1).reshape(T, K + 1)
    topk_ids = ibuf.transpose(0, 2, 1).reshape(T, K + 1)
    return topk_weights, topk_ids, router_logits


# speedup vs baseline: 3.5902x; 1.2799x over previous
import functools

import jax
import jax.numpy as jnp
from jax import lax
from jax.experimental import pallas as pl
from jax.experimental.pallas import tpu as pltpu
from jax.experimental.pallas import tpu_sc as plsc

T = 16384
K = 7
NE = 64
V = 100000
INV_ROUTED_SCALING = 1.0 / 1.5
TPW = 512

_P = (0.9998879, -0.49636775, 0.30467087, -0.15602694, 0.041064072)

try:
    _INFO = plsc.get_sparse_core_info()
    _NC = _INFO.num_cores
    _NS = _INFO.num_subcores
except Exception:
    _NC, _NS = 2, 16


def _sqrt_softplus(x):
    e = jnp.exp(-jnp.abs(x))
    acc = jnp.full((16,), _P[4], jnp.float32)
    for k in range(3, -1, -1):
        acc = acc * e + _P[k]
    sp = jnp.maximum(x, 0.0) + acc * e
    b = plsc.bitcast(sp, jnp.int32)
    q = plsc.bitcast(0x5F3759DF - lax.shift_right_logical(b, 1), jnp.float32)
    for _ in range(2):
        q = q * (1.5 - 0.5 * sp * q * q)
    return sp * q


@functools.cache
def _build_route():
    @functools.partial(
        pl.kernel,
        mesh=plsc.VectorSubcoreMesh(core_axis_name="c", subcore_axis_name="s"),
        out_type=[
            jax.ShapeDtypeStruct((T // 128, K + 1, 128), jnp.float32),
            jax.ShapeDtypeStruct((T // 128, K + 1, 128), jnp.int32),
        ],
        scratch_types=[
            pltpu.VMEM((TPW,), jnp.int32),
            pltpu.VMEM((K, 4, 128), jnp.int32),
            pltpu.VMEM((K, 4, 128), jnp.int32),
            pltpu.VMEM((NE, TPW), jnp.float32),
            pltpu.VMEM((4, K + 1, 128), jnp.float32),
            pltpu.VMEM((4, K + 1, 128), jnp.int32),
            pltpu.SemaphoreType.DMA,
            pltpu.SemaphoreType.DMA,
        ],
        compiler_params=pltpu.CompilerParams(
            use_tc_tiling_on_sc=False, needs_layout_passes=False),
    )
    def _route(ids_hbm, tab_hbm, lg_hbm, w_hbm, i_hbm,
               idx_v, widx_v, eid_v, lg_v, ow_v, oi_v, sem, sem2):
        wid = lax.axis_index("s") * _NC + lax.axis_index("c")
        base = wid * TPW
        lg_cp = pltpu.async_copy(
            lg_hbm.at[:, pl.ds(base, TPW)], lg_v, sem2)
        pltpu.sync_copy(ids_hbm.at[pl.ds(base, TPW)], idx_v)

        for c in range(4):
            def wbody(g8, carry, c=c):
                tok = idx_v[pl.ds(c * 128 + g8 * 16, 16)]
                for j in range(K):
                    widx_v[j, c, pl.ds(g8 * 16, 16)] = tok + j * V
                return carry
            lax.fori_loop(0, 8, wbody, 0)

        cps = [[pltpu.async_copy(tab_hbm.at[widx_v.at[j, c]],
                                 eid_v.at[j, c], sem)
                for j in range(K)] for c in range(4)]
        lg_cp.wait()

        for c in range(4):
            for cp in cps[c]:
                cp.wait()

            def ebody(g8, carry, c=c):
                t16 = jnp.arange(16, dtype=jnp.int32) + c * 128 + g8 * 16
                eids, ws = [], []
                for j in range(K):
                    eid = eid_v[j, c, pl.ds(g8 * 16, 16)]
                    x = plsc.load_gather(lg_v, [eid, t16])
                    eids.append(eid)
                    ws.append(_sqrt_softplus(x))
                wsum = ws[0]
                for w in ws[1:]:
                    wsum = wsum + w
                inv = 1.0 / wsum
                sn = jnp.zeros((16,), jnp.float32)
                for j in range(K):
                    wn = ws[j] * inv
                    sn = sn + wn
                    ow_v[c, j, pl.ds(g8 * 16, 16)] = wn
                    oi_v[c, j, pl.ds(g8 * 16, 16)] = eids[j]
                ow_v[c, K, pl.ds(g8 * 16, 16)] = sn * INV_ROUTED_SCALING
                oi_v[c, K, pl.ds(g8 * 16, 16)] = jnp.full((16,), NE,
                                                          jnp.int32)
                return carry
            lax.fori_loop(0, 8, ebody, 0)

        pltpu.sync_copy(ow_v, w_hbm.at[pl.ds(base // 128, 4)])
        pltpu.sync_copy(oi_v, i_hbm.at[pl.ds(base // 128, 4)])

    return _route


def kernel(hidden_states, router_logits, input_ids, tid2eid):
    del hidden_states
    tab_flat = tid2eid.T.reshape(V * K)
    lg_t = router_logits.T
    wbuf, ibuf = _build_route()(input_ids, tab_flat, lg_t)
    topk_weights = wbuf.transpose(0, 2, 1).reshape(T, K + 1)
    topk_ids = ibuf.transpose(0, 2, 1).reshape(T, K + 1)
    return topk_weights, topk_ids, router_logits
